# baseline (speedup 1.0000x reference)
import functools

import jax
import jax.numpy as jnp
from jax import lax
from jax.experimental import pallas as pl
from jax.experimental.pallas import tpu as pltpu
from jax.experimental.pallas import tpu_sc as plsc

F32 = jnp.float32
NC = 2
NS = 16
LANES = 16


def _h0_pre_body(nf_ref, wn_ref, bn_ref, w1_ref, oh_ref, oa_ref, ob_ref):
    hh = wn_ref.shape[1]
    h = jnp.dot(nf_ref[...], wn_ref[...], preferred_element_type=F32) + bn_ref[...]
    oh_ref[...] = h
    oa_ref[...] = jnp.dot(h, w1_ref[0:hh, :], preferred_element_type=F32)
    ob_ref[...] = jnp.dot(h, w1_ref[hh:2 * hh, :], preferred_element_type=F32)


def _h0_pre(nf, wn, bn, w1):
    n = nf.shape[0]
    out = jax.ShapeDtypeStruct((n, wn.shape[1]), F32)
    return pl.pallas_call(
        _h0_pre_body, out_shape=(out, out, out),
    )(nf, wn, bn.reshape(1, -1), w1)


def _ec_body(ef_ref, we_ref, be_ref, c_ref, b1_ref, o_ref):
    km = jnp.dot(we_ref[...], c_ref[0], preferred_element_type=F32)
    cv = jnp.dot(be_ref[...], c_ref[0], preferred_element_type=F32) + b1_ref[0]
    o_ref[0] = jnp.dot(ef_ref[...], km, preferred_element_type=F32) + cv


def _ec_precompute(ef, we, be, c_all, b1_all, eb=4000):
    e, ed = ef.shape
    nl, _, hh = c_all.shape
    grid = (nl, e // eb)
    return pl.pallas_call(
        _ec_body,
        grid=grid,
        in_specs=[
            pl.BlockSpec((eb, ed), lambda l, i: (i, 0)),
            pl.BlockSpec((ed, hh), lambda l, i: (0, 0)),
            pl.BlockSpec((1, hh), lambda l, i: (0, 0)),
            pl.BlockSpec((1, hh, hh), lambda l, i: (l, 0, 0)),
            pl.BlockSpec((1, 1, hh), lambda l, i: (l, 0, 0)),
        ],
        out_specs=pl.BlockSpec((1, eb, hh), lambda l, i: (l, i, 0)),
        out_shape=jax.ShapeDtypeStruct((nl, e, hh), F32),
    )(ef, we, be.reshape(1, -1), c_all, b1_all)


def _update(h_ref, p_ref, c_ref, w2_ref, b2_ref, uw_ref, ub_ref, g_ref,
            bb_ref):
    h = h_ref[...]
    n, hh = h.shape
    rl = p_ref[0] + p_ref[1]
    cnt = c_ref[0, :, 0:1] + c_ref[1, :, 0:1]
    agg = jnp.dot(rl, w2_ref[...], preferred_element_type=F32) + cnt * b2_ref[...]
    z = (jnp.dot(h, uw_ref[0:hh, :], preferred_element_type=F32)
         + jnp.dot(agg, uw_ref[hh:2 * hh, :], preferred_element_type=F32)
         + ub_ref[...])
    t = h + jnp.maximum(z, 0.0)
    mu = jnp.mean(t, axis=1, keepdims=True)
    var = jnp.mean((t - mu) * (t - mu), axis=1, keepdims=True)
    return (t - mu) / jnp.sqrt(var + 1e-5) * g_ref[...] + bb_ref[...]


def _post_pre_body(h_ref, p_ref, c_ref, w2_ref, b2_ref, uw_ref, ub_ref, g_ref,
                   bb_ref, w1_ref, oh_ref, oa_ref, ob_ref):
    hh = h_ref.shape[1]
    hn = _update(h_ref, p_ref, c_ref, w2_ref, b2_ref, uw_ref, ub_ref, g_ref,
                 bb_ref)
    oh_ref[...] = hn
    oa_ref[...] = jnp.dot(hn, w1_ref[0:hh, :], preferred_element_type=F32)
    ob_ref[...] = jnp.dot(hn, w1_ref[hh:2 * hh, :], preferred_element_type=F32)


def _post_pre(h, parts, cparts, lp, w1_next, nb=2000):
    n, hh = h.shape
    cw = cparts.shape[1]
    out = jax.ShapeDtypeStruct(h.shape, F32)
    full = lambda shape: pl.BlockSpec(shape, lambda i: tuple(0 for _ in shape))
    rowblk = pl.BlockSpec((nb, hh), lambda i: (i, 0))
    return pl.pallas_call(
        _post_pre_body,
        grid=(n // nb,),
        in_specs=[
            rowblk,
            pl.BlockSpec((2, nb, hh), lambda i: (0, i, 0)),
            pl.BlockSpec((2, nb, cw), lambda i: (0, i, 0)),
            full((hh, hh)), full((1, hh)), full((2 * hh, hh)), full((1, hh)),
            full((1, hh)), full((1, hh)), full((2 * hh, hh)),
        ],
        out_specs=(rowblk, rowblk, rowblk),
        out_shape=(out, out, out),
    )(h, parts.reshape(2, n, hh), cparts.reshape(2, n, cw), lp['msg_W2'],
      lp['msg_b2'].reshape(1, -1), lp['upd_W'], lp['upd_b'].reshape(1, -1),
      lp['ln_g'].reshape(1, -1), lp['ln_b'].reshape(1, -1), w1_next)


def _post_readout_body(h_ref, p_ref, c_ref, w2_ref, b2_ref, uw_ref, ub_ref,
                       g_ref, bb_ref, rw1_ref, rb1_ref, rw2_ref, rb2_ref,
                       rw3_ref, rb3_ref, o_ref):
    n, hh = h_ref.shape
    h = _update(h_ref, p_ref, c_ref, w2_ref, b2_ref, uw_ref, ub_ref, g_ref,
                bb_ref)
    hm = jnp.sum(h, axis=0, keepdims=True) * (1.0 / (n + 1))
    hx = jnp.max(h, axis=0, keepdims=True)
    r1 = jnp.maximum(
        jnp.dot(hm, rw1_ref[0:hh, :], preferred_element_type=F32)
        + jnp.dot(hx, rw1_ref[hh:2 * hh, :], preferred_element_type=F32)
        + rb1_ref[...], 0.0)
    r2 = jnp.maximum(
        jnp.dot(r1, rw2_ref[...], preferred_element_type=F32) + rb2_ref[...],
        0.0)
    o_ref[...] = jnp.sum(r2 * rw3_ref[...], axis=1, keepdims=True) + rb3_ref[...]


def _post_readout(h, parts, cparts, lp, params):
    n, hh = h.shape
    return pl.pallas_call(
        _post_readout_body, out_shape=jax.ShapeDtypeStruct((1, 1), F32),
    )(h, parts.reshape(2, n, hh), cparts.reshape(2, n, cparts.shape[1]),
      lp['msg_W2'], lp['msg_b2'].reshape(1, -1),
      lp['upd_W'], lp['upd_b'].reshape(1, -1), lp['ln_g'].reshape(1, -1),
      lp['ln_b'].reshape(1, -1), params['ro_W1'],
      params['ro_b1'].reshape(1, -1), params['ro_W2'],
      params['ro_b2'].reshape(1, -1), params['ro_W3'].reshape(1, -1),
      params['ro_b3'].reshape(1, 1))


_MESH = plsc.VectorSubcoreMesh(core_axis_name="c", subcore_axis_name="s")


_K = 80
_GRP = 5


def _edge_sc(hA, hB, ec, li, ei_main):
    n, hh = hA.shape
    nbody = ei_main.shape[1]
    nchunk = nbody * _GRP
    epw = nchunk * _K
    rps = (n // NS) & ~7
    ntail = n - NS * rps
    zch = _K
    nz = rps // zch
    zrem = rps - nz * zch
    nvec = hh // LANES

    @functools.partial(
        pl.kernel,
        mesh=_MESH,
        out_type=jax.ShapeDtypeStruct((2 * n, hh), F32),
        scratch_types=[
            pltpu.VMEM_SHARED((n, hh), F32),
            pltpu.VMEM((_GRP, 2, _K), jnp.int32),
            pltpu.VMEM((_K, hh), F32),
            pltpu.VMEM((_K, hh), F32),
            pltpu.VMEM((_K, hh), F32),
            pltpu.VMEM((_K, hh), F32),
            pltpu.SemaphoreType.DMA,
            pltpu.SemaphoreType.DMA,
        ],
    )
    def body(ha_hbm, hb_hbm, ec_hbm, eim_hbm, out_hbm,
             agg_sh, ib, a_v, b_v, e_v, r_v, gsem, ssem):
        c = lax.axis_index("c")
        s = lax.axis_index("s")
        wid = c * NS + s

        @pl.loop(0, zch)
        def _(r):
            for j in range(nvec):
                r_v[r, pl.ds(j * LANES, LANES)] = jnp.zeros((LANES,), F32)

        @pl.loop(0, nz)
        def _(kz):
            pltpu.sync_copy(r_v.at[pl.ds(0, zch)],
                            agg_sh.at[pl.ds(s * rps + kz * zch, zch)])

        if zrem:
            pltpu.sync_copy(r_v.at[pl.ds(0, zrem)],
                            agg_sh.at[pl.ds(s * rps + nz * zch, zrem)])

        @pl.when(s == NS - 1)
        def _():
            pltpu.sync_copy(r_v.at[pl.ds(0, ntail)],
                            agg_sh.at[pl.ds(NS * rps, ntail)])

        plsc.subcore_barrier()

        @pl.loop(0, nbody)
        def _(g):
            base = g * _GRP
            pltpu.sync_copy(eim_hbm.at[wid, g], ib)

            for j in range(_GRP):
                cj = base + j
                ca = pltpu.async_copy(ha_hbm.at[ib.at[j, 0]], a_v, gsem)
                cb = pltpu.async_copy(hb_hbm.at[ib.at[j, 1]], b_v, gsem)
                ce = pltpu.async_copy(
                    ec_hbm.at[li, pl.ds(wid * epw + cj * _K, _K)], e_v, gsem)
                ce.wait()
                ca.wait()
                cb.wait()

                @pl.loop(0, _K)
                def _(r):
                    for q in range(nvec):
                        sl = pl.ds(q * LANES, LANES)
                        r_v[r, sl] = jnp.maximum(
                            a_v[r, sl] + b_v[r, sl] + e_v[r, sl], 0.0)

                pltpu.sync_copy(r_v, agg_sh.at[ib.at[j, 1]], add=True)

        plsc.subcore_barrier()

        pltpu.sync_copy(agg_sh.at[pl.ds(s * rps, rps)],
                        out_hbm.at[pl.ds(c * n + s * rps, rps)])

        @pl.when(s == NS - 1)
        def _():
            pltpu.sync_copy(agg_sh.at[pl.ds(NS * rps, ntail)],
                            out_hbm.at[pl.ds(c * n + NS * rps, ntail)])

    return body(hA, hB, ec, ei_main)


def _counts_sc(dst, n):
    e = dst.shape[0]
    epw = e // (NC * NS)
    k = 80
    nchunk = epw // k
    w = LANES
    rps = (n // NS) & ~7
    tail = n - NS * rps

    @functools.partial(
        pl.kernel,
        mesh=_MESH,
        out_type=jax.ShapeDtypeStruct((2 * n, w), F32),
        scratch_types=[
            pltpu.VMEM_SHARED((n, w), F32),
            pltpu.VMEM((k,), jnp.int32),
            pltpu.VMEM((k, w), F32),
            pltpu.VMEM((rps, w), F32),
        ],
    )
    def body(dst_hbm, out_hbm, cnt_sh, di, ones_v, z_v):
        c = lax.axis_index("c")
        s = lax.axis_index("s")
        wid = c * NS + s

        @pl.loop(0, rps)
        def _(r):
            z_v[r, pl.ds(0, LANES)] = jnp.zeros((LANES,), F32)

        @pl.loop(0, k)
        def _(r):
            ones_v[r, pl.ds(0, LANES)] = jnp.ones((LANES,), F32)

        pltpu.sync_copy(z_v, cnt_sh.at[pl.ds(s * rps, rps)])

        @pl.when(s == NS - 1)
        def _():
            pltpu.sync_copy(z_v.at[pl.ds(0, tail)],
                            cnt_sh.at[pl.ds(NS * rps, tail)])

        plsc.subcore_barrier()

        @pl.loop(0, nchunk)
        def _(i):
            pltpu.sync_copy(dst_hbm.at[pl.ds(wid * epw + i * k, k)], di)
            pltpu.sync_copy(ones_v, cnt_sh.at[di], add=True)

        plsc.subcore_barrier()

        pltpu.sync_copy(cnt_sh.at[pl.ds(s * rps, rps)],
                        out_hbm.at[pl.ds(c * n + s * rps, rps)])

        @pl.when(s == NS - 1)
        def _():
            pltpu.sync_copy(cnt_sh.at[pl.ds(NS * rps, tail)],
                            out_hbm.at[pl.ds(c * n + NS * rps, tail)])

    return body(dst)


def kernel(node_feats, edge_index, edge_feats, params):
    n = node_feats.shape[0]
    hh = params['W_node'].shape[1]
    e = edge_index.shape[1]
    src = edge_index[0]
    dst = edge_index[1]

    nw = NC * NS
    epw = e // nw
    nbody = epw // (_K * _GRP)
    src_t = src.reshape(nw, epw)
    dst_t = dst.reshape(nw, epw)
    ei_main = jnp.stack(
        [src_t.reshape(nw, nbody, _GRP, _K),
         dst_t.reshape(nw, nbody, _GRP, _K)], axis=3)

    c_all = jnp.stack([lp['msg_W1'][2 * hh:3 * hh, :] for lp in params['layers']])
    b1_all = jnp.stack([lp['msg_b1'] for lp in params['layers']])[:, None, :]

    layers = params['layers']
    h, hA, hB = _h0_pre(node_feats, params['W_node'], params['b_node'],
                        layers[0]['msg_W1'])
    ec_all = _ec_precompute(edge_feats, params['W_edge'], params['b_edge'],
                            c_all, b1_all)
    cparts = _counts_sc(dst, n)

    for li, lp in enumerate(layers):
        parts = _edge_sc(hA, hB, ec_all, li, ei_main)
        if li + 1 < len(layers):
            h, hA, hB = _post_pre(h, parts, cparts, lp,
                                  layers[li + 1]['msg_W1'])
        else:
            out = _post_readout(h, parts, cparts, lp, params)
    return out.reshape(1)


# speedup vs baseline: 1.0268x; 1.0268x over previous
import functools

import jax
import jax.numpy as jnp
from jax import lax
from jax.experimental import pallas as pl
from jax.experimental.pallas import tpu as pltpu
from jax.experimental.pallas import tpu_sc as plsc

F32 = jnp.float32
NC = 2
NS = 16
LANES = 16


def _h0_pre_body(nf_ref, wn_ref, bn_ref, w1_ref, oh_ref, oa_ref, ob_ref):
    hh = wn_ref.shape[1]
    h = jnp.dot(nf_ref[...], wn_ref[...], preferred_element_type=F32) + bn_ref[...]
    oh_ref[...] = h
    oa_ref[...] = jnp.dot(h, w1_ref[0:hh, :], preferred_element_type=F32)
    ob_ref[...] = jnp.dot(h, w1_ref[hh:2 * hh, :], preferred_element_type=F32)


def _h0_pre(nf, wn, bn, w1):
    n = nf.shape[0]
    out = jax.ShapeDtypeStruct((n, wn.shape[1]), F32)
    return pl.pallas_call(
        _h0_pre_body, out_shape=(out, out, out),
    )(nf, wn, bn.reshape(1, -1), w1)


def _ec_body(ef_ref, we_ref, be_ref, c_ref, b1_ref, o_ref):
    km = jnp.dot(we_ref[...], c_ref[0], preferred_element_type=F32)
    cv = jnp.dot(be_ref[...], c_ref[0], preferred_element_type=F32) + b1_ref[0]
    o_ref[0] = jnp.dot(ef_ref[...], km, preferred_element_type=F32) + cv


def _ec_precompute(ef, we, be, c_all, b1_all, eb=16000):
    e, ed = ef.shape
    nl, _, hh = c_all.shape
    grid = (nl, e // eb)
    return pl.pallas_call(
        _ec_body,
        grid=grid,
        in_specs=[
            pl.BlockSpec((eb, ed), lambda l, i: (i, 0)),
            pl.BlockSpec((ed, hh), lambda l, i: (0, 0)),
            pl.BlockSpec((1, hh), lambda l, i: (0, 0)),
            pl.BlockSpec((1, hh, hh), lambda l, i: (l, 0, 0)),
            pl.BlockSpec((1, 1, hh), lambda l, i: (l, 0, 0)),
        ],
        out_specs=pl.BlockSpec((1, eb, hh), lambda l, i: (l, i, 0)),
        out_shape=jax.ShapeDtypeStruct((nl, e, hh), F32),
    )(ef, we, be.reshape(1, -1), c_all, b1_all)


def _update(h_ref, p_ref, c_ref, w2_ref, b2_ref, uw_ref, ub_ref, g_ref,
            bb_ref):
    h = h_ref[...]
    n, hh = h.shape
    rl = p_ref[0] + p_ref[1]
    cnt = c_ref[0, :, 0:1] + c_ref[1, :, 0:1]
    agg = jnp.dot(rl, w2_ref[...], preferred_element_type=F32) + cnt * b2_ref[...]
    z = (jnp.dot(h, uw_ref[0:hh, :], preferred_element_type=F32)
         + jnp.dot(agg, uw_ref[hh:2 * hh, :], preferred_element_type=F32)
         + ub_ref[...])
    t = h + jnp.maximum(z, 0.0)
    mu = jnp.mean(t, axis=1, keepdims=True)
    var = jnp.mean((t - mu) * (t - mu), axis=1, keepdims=True)
    return (t - mu) / jnp.sqrt(var + 1e-5) * g_ref[...] + bb_ref[...]


def _post_pre_body(h_ref, p_ref, c_ref, w2_ref, b2_ref, uw_ref, ub_ref, g_ref,
                   bb_ref, w1_ref, oh_ref, oa_ref, ob_ref):
    hh = h_ref.shape[1]
    hn = _update(h_ref, p_ref, c_ref, w2_ref, b2_ref, uw_ref, ub_ref, g_ref,
                 bb_ref)
    oh_ref[...] = hn
    oa_ref[...] = jnp.dot(hn, w1_ref[0:hh, :], preferred_element_type=F32)
    ob_ref[...] = jnp.dot(hn, w1_ref[hh:2 * hh, :], preferred_element_type=F32)


def _post_pre(h, parts, cparts, lp, w1_next, nb=None):
    nb = h.shape[0] if nb is None else nb
    n, hh = h.shape
    cw = cparts.shape[1]
    out = jax.ShapeDtypeStruct(h.shape, F32)
    full = lambda shape: pl.BlockSpec(shape, lambda i: tuple(0 for _ in shape))
    rowblk = pl.BlockSpec((nb, hh), lambda i: (i, 0))
    return pl.pallas_call(
        _post_pre_body,
        grid=(n // nb,),
        in_specs=[
            rowblk,
            pl.BlockSpec((2, nb, hh), lambda i: (0, i, 0)),
            pl.BlockSpec((2, nb, cw), lambda i: (0, i, 0)),
            full((hh, hh)), full((1, hh)), full((2 * hh, hh)), full((1, hh)),
            full((1, hh)), full((1, hh)), full((2 * hh, hh)),
        ],
        out_specs=(rowblk, rowblk, rowblk),
        out_shape=(out, out, out),
    )(h, parts.reshape(2, n, hh), cparts.reshape(2, n, cw), lp['msg_W2'],
      lp['msg_b2'].reshape(1, -1), lp['upd_W'], lp['upd_b'].reshape(1, -1),
      lp['ln_g'].reshape(1, -1), lp['ln_b'].reshape(1, -1), w1_next)


def _post_readout_body(h_ref, p_ref, c_ref, w2_ref, b2_ref, uw_ref, ub_ref,
                       g_ref, bb_ref, rw1_ref, rb1_ref, rw2_ref, rb2_ref,
                       rw3_ref, rb3_ref, o_ref):
    n, hh = h_ref.shape
    h = _update(h_ref, p_ref, c_ref, w2_ref, b2_ref, uw_ref, ub_ref, g_ref,
                bb_ref)
    hm = jnp.sum(h, axis=0, keepdims=True) * (1.0 / (n + 1))
    hx = jnp.max(h, axis=0, keepdims=True)
    r1 = jnp.maximum(
        jnp.dot(hm, rw1_ref[0:hh, :], preferred_element_type=F32)
        + jnp.dot(hx, rw1_ref[hh:2 * hh, :], preferred_element_type=F32)
        + rb1_ref[...], 0.0)
    r2 = jnp.maximum(
        jnp.dot(r1, rw2_ref[...], preferred_element_type=F32) + rb2_ref[...],
        0.0)
    o_ref[...] = jnp.sum(r2 * rw3_ref[...], axis=1, keepdims=True) + rb3_ref[...]


def _post_readout(h, parts, cparts, lp, params):
    n, hh = h.shape
    return pl.pallas_call(
        _post_readout_body, out_shape=jax.ShapeDtypeStruct((1, 1), F32),
    )(h, parts.reshape(2, n, hh), cparts.reshape(2, n, cparts.shape[1]),
      lp['msg_W2'], lp['msg_b2'].reshape(1, -1),
      lp['upd_W'], lp['upd_b'].reshape(1, -1), lp['ln_g'].reshape(1, -1),
      lp['ln_b'].reshape(1, -1), params['ro_W1'],
      params['ro_b1'].reshape(1, -1), params['ro_W2'],
      params['ro_b2'].reshape(1, -1), params['ro_W3'].reshape(1, -1),
      params['ro_b3'].reshape(1, 1))


_MESH = plsc.VectorSubcoreMesh(core_axis_name="c", subcore_axis_name="s")


_K = 80
_GRP = 5


def _edge_sc(hA, hB, ec, li, ei_main):
    n, hh = hA.shape
    nbody = ei_main.shape[1]
    nchunk = nbody * _GRP
    epw = nchunk * _K
    rps = (n // NS) & ~7
    ntail = n - NS * rps
    zch = _K
    nz = rps // zch
    zrem = rps - nz * zch
    nvec = hh // LANES

    @functools.partial(
        pl.kernel,
        mesh=_MESH,
        out_type=jax.ShapeDtypeStruct((2 * n, hh), F32),
        scratch_types=[
            pltpu.VMEM_SHARED((n, hh), F32),
            pltpu.VMEM((_GRP, 2, _K), jnp.int32),
            pltpu.VMEM((_K, hh), F32),
            pltpu.VMEM((_K, hh), F32),
            pltpu.VMEM((_K, hh), F32),
            pltpu.VMEM((_K, hh), F32),
            pltpu.SemaphoreType.DMA,
            pltpu.SemaphoreType.DMA,
        ],
    )
    def body(ha_hbm, hb_hbm, ec_hbm, eim_hbm, out_hbm,
             agg_sh, ib, a_v, b_v, e_v, r_v, gsem, ssem):
        c = lax.axis_index("c")
        s = lax.axis_index("s")
        wid = c * NS + s

        @pl.loop(0, zch)
        def _(r):
            for j in range(nvec):
                r_v[r, pl.ds(j * LANES, LANES)] = jnp.zeros((LANES,), F32)

        @pl.loop(0, nz)
        def _(kz):
            pltpu.sync_copy(r_v.at[pl.ds(0, zch)],
                            agg_sh.at[pl.ds(s * rps + kz * zch, zch)])

        if zrem:
            pltpu.sync_copy(r_v.at[pl.ds(0, zrem)],
                            agg_sh.at[pl.ds(s * rps + nz * zch, zrem)])

        @pl.when(s == NS - 1)
        def _():
            pltpu.sync_copy(r_v.at[pl.ds(0, ntail)],
                            agg_sh.at[pl.ds(NS * rps, ntail)])

        plsc.subcore_barrier()

        @pl.loop(0, nbody)
        def _(g):
            base = g * _GRP
            pltpu.sync_copy(eim_hbm.at[wid, g], ib)

            for j in range(_GRP):
                cj = base + j
                ca = pltpu.async_copy(ha_hbm.at[ib.at[j, 0]], a_v, gsem)
                cb = pltpu.async_copy(hb_hbm.at[ib.at[j, 1]], b_v, gsem)
                ce = pltpu.async_copy(
                    ec_hbm.at[li, pl.ds(wid * epw + cj * _K, _K)], e_v, gsem)
                ce.wait()
                ca.wait()
                cb.wait()

                @pl.loop(0, _K)
                def _(r):
                    for q in range(nvec):
                        sl = pl.ds(q * LANES, LANES)
                        r_v[r, sl] = jnp.maximum(
                            a_v[r, sl] + b_v[r, sl] + e_v[r, sl], 0.0)

                pltpu.sync_copy(r_v, agg_sh.at[ib.at[j, 1]], add=True)

        plsc.subcore_barrier()

        pltpu.sync_copy(agg_sh.at[pl.ds(s * rps, rps)],
                        out_hbm.at[pl.ds(c * n + s * rps, rps)])

        @pl.when(s == NS - 1)
        def _():
            pltpu.sync_copy(agg_sh.at[pl.ds(NS * rps, ntail)],
                            out_hbm.at[pl.ds(c * n + NS * rps, ntail)])

    return body(hA, hB, ec, ei_main)


def _counts_sc(dst, n):
    e = dst.shape[0]
    epw = e // (NC * NS)
    k = 80
    nchunk = epw // k
    w = LANES
    rps = (n // NS) & ~7
    tail = n - NS * rps

    @functools.partial(
        pl.kernel,
        mesh=_MESH,
        out_type=jax.ShapeDtypeStruct((2 * n, w), F32),
        scratch_types=[
            pltpu.VMEM_SHARED((n, w), F32),
            pltpu.VMEM((k,), jnp.int32),
            pltpu.VMEM((k, w), F32),
            pltpu.VMEM((rps, w), F32),
        ],
    )
    def body(dst_hbm, out_hbm, cnt_sh, di, ones_v, z_v):
        c = lax.axis_index("c")
        s = lax.axis_index("s")
        wid = c * NS + s

        @pl.loop(0, rps)
        def _(r):
            z_v[r, pl.ds(0, LANES)] = jnp.zeros((LANES,), F32)

        @pl.loop(0, k)
        def _(r):
            ones_v[r, pl.ds(0, LANES)] = jnp.ones((LANES,), F32)

        pltpu.sync_copy(z_v, cnt_sh.at[pl.ds(s * rps, rps)])

        @pl.when(s == NS - 1)
        def _():
            pltpu.sync_copy(z_v.at[pl.ds(0, tail)],
                            cnt_sh.at[pl.ds(NS * rps, tail)])

        plsc.subcore_barrier()

        @pl.loop(0, nchunk)
        def _(i):
            pltpu.sync_copy(dst_hbm.at[pl.ds(wid * epw + i * k, k)], di)
            pltpu.sync_copy(ones_v, cnt_sh.at[di], add=True)

        plsc.subcore_barrier()

        pltpu.sync_copy(cnt_sh.at[pl.ds(s * rps, rps)],
                        out_hbm.at[pl.ds(c * n + s * rps, rps)])

        @pl.when(s == NS - 1)
        def _():
            pltpu.sync_copy(cnt_sh.at[pl.ds(NS * rps, tail)],
                            out_hbm.at[pl.ds(c * n + NS * rps, tail)])

    return body(dst)


def kernel(node_feats, edge_index, edge_feats, params):
    n = node_feats.shape[0]
    hh = params['W_node'].shape[1]
    e = edge_index.shape[1]
    src = edge_index[0]
    dst = edge_index[1]

    nw = NC * NS
    epw = e // nw
    nbody = epw // (_K * _GRP)
    src_t = src.reshape(nw, epw)
    dst_t = dst.reshape(nw, epw)
    ei_main = jnp.stack(
        [src_t.reshape(nw, nbody, _GRP, _K),
         dst_t.reshape(nw, nbody, _GRP, _K)], axis=3)

    c_all = jnp.stack([lp['msg_W1'][2 * hh:3 * hh, :] for lp in params['layers']])
    b1_all = jnp.stack([lp['msg_b1'] for lp in params['layers']])[:, None, :]

    layers = params['layers']
    h, hA, hB = _h0_pre(node_feats, params['W_node'], params['b_node'],
                        layers[0]['msg_W1'])
    ec_all = _ec_precompute(edge_feats, params['W_edge'], params['b_edge'],
                            c_all, b1_all)
    cparts = _counts_sc(dst, n)

    for li, lp in enumerate(layers):
        parts = _edge_sc(hA, hB, ec_all, li, ei_main)
        if li + 1 < len(layers):
            h, hA, hB = _post_pre(h, parts, cparts, lp,
                                  layers[li + 1]['msg_W1'])
        else:
            out = _post_readout(h, parts, cparts, lp, params)
    return out.reshape(1)


# speedup vs baseline: 1.0975x; 1.0689x over previous
import functools

import jax
import jax.numpy as jnp
from jax import lax
from jax.experimental import pallas as pl
from jax.experimental.pallas import tpu as pltpu
from jax.experimental.pallas import tpu_sc as plsc

F32 = jnp.float32
NC = 2
NS = 16
LANES = 16


def _h0_pre_body(nf_ref, wn_ref, bn_ref, w1_ref, oh_ref, oa_ref, ob_ref):
    hh = wn_ref.shape[1]
    h = jnp.dot(nf_ref[...], wn_ref[...], preferred_element_type=F32) + bn_ref[...]
    oh_ref[...] = h
    oa_ref[...] = jnp.dot(h, w1_ref[0:hh, :], preferred_element_type=F32)
    ob_ref[...] = jnp.dot(h, w1_ref[hh:2 * hh, :], preferred_element_type=F32)


def _h0_pre(nf, wn, bn, w1):
    n = nf.shape[0]
    out = jax.ShapeDtypeStruct((n, wn.shape[1]), F32)
    return pl.pallas_call(
        _h0_pre_body, out_shape=(out, out, out),
    )(nf, wn, bn.reshape(1, -1), w1)


def _ec_body(ef_ref, we_ref, be_ref, c_ref, b1_ref, o_ref):
    km = jnp.dot(we_ref[...], c_ref[0], preferred_element_type=F32)
    cv = jnp.dot(be_ref[...], c_ref[0], preferred_element_type=F32) + b1_ref[0]
    o_ref[0] = jnp.dot(ef_ref[...], km, preferred_element_type=F32) + cv


def _ec_precompute(ef, we, be, c_all, b1_all, eb=16000):
    e, ed = ef.shape
    nl, _, hh = c_all.shape
    grid = (nl, e // eb)
    return pl.pallas_call(
        _ec_body,
        grid=grid,
        in_specs=[
            pl.BlockSpec((eb, ed), lambda l, i: (i, 0)),
            pl.BlockSpec((ed, hh), lambda l, i: (0, 0)),
            pl.BlockSpec((1, hh), lambda l, i: (0, 0)),
            pl.BlockSpec((1, hh, hh), lambda l, i: (l, 0, 0)),
            pl.BlockSpec((1, 1, hh), lambda l, i: (l, 0, 0)),
        ],
        out_specs=pl.BlockSpec((1, eb, hh), lambda l, i: (l, i, 0)),
        out_shape=jax.ShapeDtypeStruct((nl, e, hh), F32),
    )(ef, we, be.reshape(1, -1), c_all, b1_all)


def _update(h_ref, p_ref, c_ref, w2_ref, b2_ref, uw_ref, ub_ref, g_ref,
            bb_ref):
    h = h_ref[...]
    n, hh = h.shape
    rl = p_ref[0] + p_ref[1]
    cnt = c_ref[0, :, 0:1] + c_ref[1, :, 0:1]
    agg = jnp.dot(rl, w2_ref[...], preferred_element_type=F32) + cnt * b2_ref[...]
    z = (jnp.dot(h, uw_ref[0:hh, :], preferred_element_type=F32)
         + jnp.dot(agg, uw_ref[hh:2 * hh, :], preferred_element_type=F32)
         + ub_ref[...])
    t = h + jnp.maximum(z, 0.0)
    mu = jnp.mean(t, axis=1, keepdims=True)
    var = jnp.mean((t - mu) * (t - mu), axis=1, keepdims=True)
    return (t - mu) / jnp.sqrt(var + 1e-5) * g_ref[...] + bb_ref[...]


def _post_pre_body(h_ref, p_ref, c_ref, w2_ref, b2_ref, uw_ref, ub_ref, g_ref,
                   bb_ref, w1_ref, oh_ref, oa_ref, ob_ref):
    hh = h_ref.shape[1]
    hn = _update(h_ref, p_ref, c_ref, w2_ref, b2_ref, uw_ref, ub_ref, g_ref,
                 bb_ref)
    oh_ref[...] = hn
    oa_ref[...] = jnp.dot(hn, w1_ref[0:hh, :], preferred_element_type=F32)
    ob_ref[...] = jnp.dot(hn, w1_ref[hh:2 * hh, :], preferred_element_type=F32)


def _post_pre(h, parts, cparts, lp, w1_next, nb=None):
    nb = h.shape[0] if nb is None else nb
    n, hh = h.shape
    cw = cparts.shape[1]
    out = jax.ShapeDtypeStruct(h.shape, F32)
    full = lambda shape: pl.BlockSpec(shape, lambda i: tuple(0 for _ in shape))
    rowblk = pl.BlockSpec((nb, hh), lambda i: (i, 0))
    return pl.pallas_call(
        _post_pre_body,
        grid=(n // nb,),
        in_specs=[
            rowblk,
            pl.BlockSpec((2, nb, hh), lambda i: (0, i, 0)),
            pl.BlockSpec((2, nb, cw), lambda i: (0, i, 0)),
            full((hh, hh)), full((1, hh)), full((2 * hh, hh)), full((1, hh)),
            full((1, hh)), full((1, hh)), full((2 * hh, hh)),
        ],
        out_specs=(rowblk, rowblk, rowblk),
        out_shape=(out, out, out),
    )(h, parts.reshape(2, n, hh), cparts.reshape(2, n, cw), lp['msg_W2'],
      lp['msg_b2'].reshape(1, -1), lp['upd_W'], lp['upd_b'].reshape(1, -1),
      lp['ln_g'].reshape(1, -1), lp['ln_b'].reshape(1, -1), w1_next)


def _post_readout_body(h_ref, p_ref, c_ref, w2_ref, b2_ref, uw_ref, ub_ref,
                       g_ref, bb_ref, rw1_ref, rb1_ref, rw2_ref, rb2_ref,
                       rw3_ref, rb3_ref, o_ref):
    n, hh = h_ref.shape
    h = _update(h_ref, p_ref, c_ref, w2_ref, b2_ref, uw_ref, ub_ref, g_ref,
                bb_ref)
    hm = jnp.sum(h, axis=0, keepdims=True) * (1.0 / (n + 1))
    hx = jnp.max(h, axis=0, keepdims=True)
    r1 = jnp.maximum(
        jnp.dot(hm, rw1_ref[0:hh, :], preferred_element_type=F32)
        + jnp.dot(hx, rw1_ref[hh:2 * hh, :], preferred_element_type=F32)
        + rb1_ref[...], 0.0)
    r2 = jnp.maximum(
        jnp.dot(r1, rw2_ref[...], preferred_element_type=F32) + rb2_ref[...],
        0.0)
    o_ref[...] = jnp.sum(r2 * rw3_ref[...], axis=1, keepdims=True) + rb3_ref[...]


def _post_readout(h, parts, cparts, lp, params):
    n, hh = h.shape
    return pl.pallas_call(
        _post_readout_body, out_shape=jax.ShapeDtypeStruct((1, 1), F32),
    )(h, parts.reshape(2, n, hh), cparts.reshape(2, n, cparts.shape[1]),
      lp['msg_W2'], lp['msg_b2'].reshape(1, -1),
      lp['upd_W'], lp['upd_b'].reshape(1, -1), lp['ln_g'].reshape(1, -1),
      lp['ln_b'].reshape(1, -1), params['ro_W1'],
      params['ro_b1'].reshape(1, -1), params['ro_W2'],
      params['ro_b2'].reshape(1, -1), params['ro_W3'].reshape(1, -1),
      params['ro_b3'].reshape(1, 1))


_MESH = plsc.VectorSubcoreMesh(core_axis_name="c", subcore_axis_name="s")


_K = 80
_GRP = 5


def _edge_sc(hA, hB, ec, li, ei_main):
    n, hh = hA.shape
    nbody = ei_main.shape[1]
    nchunk = nbody * _GRP
    epw = nchunk * _K
    rps = (n // NS) & ~7
    ntail = n - NS * rps
    zch = _K
    nz = rps // zch
    zrem = rps - nz * zch
    nvec = hh // LANES

    @functools.partial(
        pl.kernel,
        mesh=_MESH,
        out_type=jax.ShapeDtypeStruct((2 * n, hh), F32),
        scratch_types=[
            pltpu.VMEM_SHARED((n, hh), F32),
            pltpu.VMEM((_GRP, 2, _K), jnp.int32),
            pltpu.VMEM((_K, hh), F32),
            pltpu.VMEM((_K, hh), F32),
            pltpu.VMEM((_K, hh), F32),
            pltpu.VMEM((_K, hh), F32),
            pltpu.SemaphoreType.DMA,
            pltpu.SemaphoreType.DMA,
        ],
    )
    def body(ha_hbm, hb_hbm, ec_hbm, eim_hbm, out_hbm,
             agg_sh, ib, a_v, b_v, e_v, r_v, gsem, ssem):
        c = lax.axis_index("c")
        s = lax.axis_index("s")
        wid = c * NS + s

        @pl.loop(0, zch)
        def _(r):
            for j in range(nvec):
                r_v[r, pl.ds(j * LANES, LANES)] = jnp.zeros((LANES,), F32)

        @pl.loop(0, nz)
        def _(kz):
            pltpu.sync_copy(r_v.at[pl.ds(0, zch)],
                            agg_sh.at[pl.ds(s * rps + kz * zch, zch)])

        if zrem:
            pltpu.sync_copy(r_v.at[pl.ds(0, zrem)],
                            agg_sh.at[pl.ds(s * rps + nz * zch, zrem)])

        @pl.when(s == NS - 1)
        def _():
            pltpu.sync_copy(r_v.at[pl.ds(0, ntail)],
                            agg_sh.at[pl.ds(NS * rps, ntail)])

        plsc.subcore_barrier()

        def scatter_wait(j):
            pltpu.make_async_copy(r_v, agg_sh.at[ib.at[j, 1]], ssem).wait()

        @pl.loop(0, nbody)
        def _(g):
            base = g * _GRP

            @pl.when(g > 0)
            def _():
                scatter_wait(_GRP - 1)

            pltpu.sync_copy(eim_hbm.at[wid, g], ib)

            for j in range(_GRP):
                cj = base + j
                ca = pltpu.async_copy(ha_hbm.at[ib.at[j, 0]], a_v, gsem)
                cb = pltpu.async_copy(hb_hbm.at[ib.at[j, 1]], b_v, gsem)
                ce = pltpu.async_copy(
                    ec_hbm.at[li, pl.ds(wid * epw + cj * _K, _K)], e_v, gsem)
                ce.wait()
                ca.wait()
                cb.wait()
                if j > 0:
                    scatter_wait(j - 1)

                @pl.loop(0, _K)
                def _(r):
                    for q in range(nvec):
                        sl = pl.ds(q * LANES, LANES)
                        r_v[r, sl] = jnp.maximum(
                            a_v[r, sl] + b_v[r, sl] + e_v[r, sl], 0.0)

                pltpu.async_copy(r_v, agg_sh.at[ib.at[j, 1]], ssem, add=True)

        scatter_wait(_GRP - 1)

        plsc.subcore_barrier()

        pltpu.sync_copy(agg_sh.at[pl.ds(s * rps, rps)],
                        out_hbm.at[pl.ds(c * n + s * rps, rps)])

        @pl.when(s == NS - 1)
        def _():
            pltpu.sync_copy(agg_sh.at[pl.ds(NS * rps, ntail)],
                            out_hbm.at[pl.ds(c * n + NS * rps, ntail)])

    return body(hA, hB, ec, ei_main)


def _counts_sc(dst, n):
    e = dst.shape[0]
    epw = e // (NC * NS)
    k = 80
    nchunk = epw // k
    w = LANES
    rps = (n // NS) & ~7
    tail = n - NS * rps

    @functools.partial(
        pl.kernel,
        mesh=_MESH,
        out_type=jax.ShapeDtypeStruct((2 * n, w), F32),
        scratch_types=[
            pltpu.VMEM_SHARED((n, w), F32),
            pltpu.VMEM((k,), jnp.int32),
            pltpu.VMEM((k, w), F32),
            pltpu.VMEM((rps, w), F32),
        ],
    )
    def body(dst_hbm, out_hbm, cnt_sh, di, ones_v, z_v):
        c = lax.axis_index("c")
        s = lax.axis_index("s")
        wid = c * NS + s

        @pl.loop(0, rps)
        def _(r):
            z_v[r, pl.ds(0, LANES)] = jnp.zeros((LANES,), F32)

        @pl.loop(0, k)
        def _(r):
            ones_v[r, pl.ds(0, LANES)] = jnp.ones((LANES,), F32)

        pltpu.sync_copy(z_v, cnt_sh.at[pl.ds(s * rps, rps)])

        @pl.when(s == NS - 1)
        def _():
            pltpu.sync_copy(z_v.at[pl.ds(0, tail)],
                            cnt_sh.at[pl.ds(NS * rps, tail)])

        plsc.subcore_barrier()

        @pl.loop(0, nchunk)
        def _(i):
            pltpu.sync_copy(dst_hbm.at[pl.ds(wid * epw + i * k, k)], di)
            pltpu.sync_copy(ones_v, cnt_sh.at[di], add=True)

        plsc.subcore_barrier()

        pltpu.sync_copy(cnt_sh.at[pl.ds(s * rps, rps)],
                        out_hbm.at[pl.ds(c * n + s * rps, rps)])

        @pl.when(s == NS - 1)
        def _():
            pltpu.sync_copy(cnt_sh.at[pl.ds(NS * rps, tail)],
                            out_hbm.at[pl.ds(c * n + NS * rps, tail)])

    return body(dst)


def kernel(node_feats, edge_index, edge_feats, params):
    n = node_feats.shape[0]
    hh = params['W_node'].shape[1]
    e = edge_index.shape[1]
    src = edge_index[0]
    dst = edge_index[1]

    nw = NC * NS
    epw = e // nw
    nbody = epw // (_K * _GRP)
    src_t = src.reshape(nw, epw)
    dst_t = dst.reshape(nw, epw)
    ei_main = jnp.stack(
        [src_t.reshape(nw, nbody, _GRP, _K),
         dst_t.reshape(nw, nbody, _GRP, _K)], axis=3)

    c_all = jnp.stack([lp['msg_W1'][2 * hh:3 * hh, :] for lp in params['layers']])
    b1_all = jnp.stack([lp['msg_b1'] for lp in params['layers']])[:, None, :]

    layers = params['layers']
    h, hA, hB = _h0_pre(node_feats, params['W_node'], params['b_node'],
                        layers[0]['msg_W1'])
    ec_all = _ec_precompute(edge_feats, params['W_edge'], params['b_edge'],
                            c_all, b1_all)
    cparts = _counts_sc(dst, n)

    for li, lp in enumerate(layers):
        parts = _edge_sc(hA, hB, ec_all, li, ei_main)
        if li + 1 < len(layers):
            h, hA, hB = _post_pre(h, parts, cparts, lp,
                                  layers[li + 1]['msg_W1'])
        else:
            out = _post_readout(h, parts, cparts, lp, params)
    return out.reshape(1)
